# baseline (speedup 1.0000x reference)
import functools

import jax
import jax.numpy as jnp
from jax import lax
from jax.experimental import pallas as pl
from jax.experimental.pallas import tpu as pltpu
from jax.experimental.pallas import tpu_sc as plsc

DH = 32
K = 16
NF = 8
N = 2000
BN = 2000
NB = N // BN
BR = 1000
NRB = N // BR
TW = 128
F32 = jnp.float32
BF16 = jnp.bfloat16
INT32 = jnp.int32

_INTERP = False


def _dot(a, w_bf16):
    return jnp.dot(a.astype(BF16), w_bf16, preferred_element_type=F32)


def _gelu(z):
    return jax.nn.gelu(z)


def _embed_body(x_ref, w_ref, b_ref, t_ref):
    xb = x_ref[0]
    h = _dot(xb[:, 3:7], w_ref[...]) + b_ref[...]
    zero1 = jnp.zeros((N, 1), F32)
    zpad = jnp.zeros((N, TW - 36), F32)
    t_ref[0] = jnp.concatenate([xb[:, 0:3], zero1, h, zpad], axis=1)


def _embed(x, w_bf16, b):
    return pl.pallas_call(
        _embed_body,
        grid=(2,),
        in_specs=[
            pl.BlockSpec((1, N, 7), lambda i: (i, 0, 0)),
            pl.BlockSpec((4, DH), lambda i: (0, 0)),
            pl.BlockSpec((DH,), lambda i: (0,)),
        ],
        out_specs=pl.BlockSpec((1, N, TW), lambda i: (i, 0, 0)),
        out_shape=jax.ShapeDtypeStruct((2, N, TW), F32),
        interpret=_INTERP,
    )(x, w_bf16, b)


def _knn_body(pos_ref, post_ref, idx_ref, d_ref):
    j = pl.program_id(1)
    b = pl.program_id(0)
    xi = pos_ref[0]
    pt = post_ref[0]
    dx = xi[:, 0:1] - pt[0:1, :]
    dy = xi[:, 1:2] - pt[1:2, :]
    dz = xi[:, 2:3] - pt[2:3, :]
    d2 = (dx * dx + dy * dy) + dz * dz
    col = lax.broadcasted_iota(INT32, (BR, N), 1)
    row = lax.broadcasted_iota(INT32, (BR, N), 0) + j * BR
    d_ref[...] = jnp.where(col == row, d2 + 1e9, d2)
    for t in range(K):
        d = d_ref[...]
        m = jnp.min(d, axis=1, keepdims=True)
        am = jnp.min(jnp.where(d == m, col, jnp.int32(2 ** 30)),
                     axis=1)
        idx_ref[0, :, t] = am + b * N
        d_ref[...] = jnp.where(col == am[:, None], jnp.float32(jnp.inf), d)


def _knn(pos4, post):
    return pl.pallas_call(
        _knn_body,
        grid=(2, NRB),
        in_specs=[
            pl.BlockSpec((1, BR, 4), lambda b, j: (b, j, 0)),
            pl.BlockSpec((1, 4, N), lambda b, j: (b, 0, 0)),
        ],
        out_specs=pl.BlockSpec((1, BR, K), lambda b, j: (b, j, 0)),
        out_shape=jax.ShapeDtypeStruct((2, N, K), INT32),
        scratch_shapes=[pltpu.VMEM((BR, N), F32)],
        interpret=_INTERP,
    )(pos4, post)


def _sc_gather(table, gidx):
    nrows = 2 * K * N
    per_w = nrows // 32
    ch = 400
    nch = per_w // ch

    mesh = plsc.VectorSubcoreMesh(core_axis_name="c", subcore_axis_name="s")

    @functools.partial(
        pl.kernel,
        mesh=mesh,
        out_type=jax.ShapeDtypeStruct((nrows, TW), F32),
        scratch_types=[
            pltpu.VMEM((ch,), INT32),
            pltpu.VMEM((ch, TW), F32),
            pltpu.SemaphoreType.DMA,
        ],
    )
    def gather_k(table_hbm, gidx_hbm, out_hbm, idx_v, rows_v, sem):
        wid = lax.axis_index("s") * 2 + lax.axis_index("c")
        base = wid * per_w
        for c in range(nch):
            off = base + c * ch
            pltpu.sync_copy(gidx_hbm.at[pl.ds(off, ch)], idx_v)
            pltpu.async_copy(table_hbm.at[idx_v], rows_v, sem).wait()
            pltpu.sync_copy(rows_v, out_hbm.at[pl.ds(off, ch)])

    return gather_k(table, gidx)


def _dott(wt_bf16, x):
    return jnp.dot(wt_bf16, x.astype(BF16), preferred_element_type=F32)


def _step_body(t_ref, g_ref,
               we1, be1, we2, be2, we3, be3,
               wx1, bx1, wx2, bx2, wx3, bx3,
               wh1, bh1, wh2, bh2, wh3, bh3,
               tn_ref):
    tt = jnp.transpose(t_ref[0][:, 0:48])
    post = tt[0:4]
    ht = tt[4:36]
    gts = [jnp.transpose(g_ref[k][:, 0:48]) for k in range(K)]
    rs = []
    d2s = []
    for k in range(K):
        rt = post - gts[k][0:4]
        x2 = rt[0:1] * rt[0:1]
        y2 = rt[1:2] * rt[1:2]
        z2 = rt[2:3] * rt[2:3]
        rs.append(rt)
        d2s.append((x2 + y2) + z2)
    ang = jnp.concatenate(
        [d2s[k] * float(2.0 ** f) for k in range(K) for f in range(NF)],
        axis=0)
    sin_all = jnp.sin(ang)
    cos_all = jnp.cos(ang)
    agg = jnp.zeros((DH, BN), F32)
    acc_rw = jnp.zeros((4, BN), F32)
    for k in range(K):
        edge = jnp.concatenate(
            [ht, gts[k][4:36],
             sin_all[8 * k:8 * k + 8], cos_all[8 * k:8 * k + 8]],
            axis=0)
        z = _gelu(_dott(we1[...], edge) + be1[...])
        z = _gelu(_dott(we2[...], z) + be2[...])
        m = _gelu(_dott(we3[...], z) + be3[...])
        u = _gelu(_dott(wx1[...], m) + bx1[...])
        u = _gelu(_dott(wx2[...], u) + bx2[...])
        w = _dott(wx3[...], u) + bx3[...]
        agg = agg + m
        acc_rw = acc_rw + rs[k] * w
    posn = post + acc_rw * (1.0 / K)
    cat = jnp.concatenate([ht, agg], axis=0)
    z = _gelu(_dott(wh1[...], cat) + bh1[...])
    z = _gelu(_dott(wh2[...], z) + bh2[...])
    hn = _dott(wh3[...], z) + bh3[...]
    out = jnp.transpose(jnp.concatenate([posn, hn], axis=0))
    zpad = jnp.zeros((BN, TW - 36), F32)
    tn_ref[0] = jnp.concatenate([out, zpad], axis=1)


def _step(t, g, wflat):
    wspecs = [pl.BlockSpec(w.shape, functools.partial(
        lambda rank, b, j: (0,) * rank, w.ndim)) for w in wflat]
    return pl.pallas_call(
        _step_body,
        grid=(2, NB),
        in_specs=[
            pl.BlockSpec((1, BN, TW), lambda b, j: (b, j, 0)),
            pl.BlockSpec((K, BN, TW), lambda b, j: (b, j, 0)),
        ] + wspecs,
        out_specs=pl.BlockSpec((1, BN, TW), lambda b, j: (b, j, 0)),
        out_shape=jax.ShapeDtypeStruct((2, N, TW), F32),
        interpret=_INTERP,
    )(t, g, *wflat)


def _decode_body(t_ref, w_ref, b_ref, out_ref):
    tb = t_ref[0]
    feat = _dot(tb[:, 4:36], w_ref[...]) + b_ref[...]
    out_ref[0] = jnp.concatenate([tb[:, 0:3], feat], axis=1)


def _decode(t, w_bf16, b):
    return pl.pallas_call(
        _decode_body,
        grid=(2,),
        in_specs=[
            pl.BlockSpec((1, N, TW), lambda i: (i, 0, 0)),
            pl.BlockSpec((DH, 4), lambda i: (0, 0)),
            pl.BlockSpec((4,), lambda i: (0,)),
        ],
        out_specs=pl.BlockSpec((1, N, 7), lambda i: (i, 0, 0)),
        out_shape=jax.ShapeDtypeStruct((2, N, 7), F32),
        interpret=_INTERP,
    )(t, w_bf16, b)


def _step_weights(p):
    ws = []
    for name in ('phi_e', 'phi_x', 'phi_h'):
        for (w, b) in p[name]:
            ws.append(w.T.astype(BF16))
            ws.append(b.reshape(-1, 1))
    return ws


def kernel(x, params):
    pos = x[:, :, 0:3]
    pos4 = jnp.pad(pos, ((0, 0), (0, 0), (0, 1)))
    post = jnp.transpose(pos4, (0, 2, 1))

    t = _embed(x, params['W_embed'].astype(BF16), params['b_embed'])
    gidx = jnp.transpose(_knn(pos4, post), (0, 2, 1)).reshape(2 * K * N)

    for p in params['steps']:
        g = _sc_gather(t.reshape(2 * N, TW), gidx).reshape(2 * K, N, TW)
        t = _step(t, g, _step_weights(p))

    return _decode(t, params['W_dec'].astype(BF16), params['b_dec'])


# speedup vs baseline: 1.0692x; 1.0692x over previous
import functools

import jax
import jax.numpy as jnp
from jax import lax
from jax.experimental import pallas as pl
from jax.experimental.pallas import tpu as pltpu
from jax.experimental.pallas import tpu_sc as plsc

DH = 32
K = 16
NF = 8
N = 2000
BN = 1000
NB = N // BN
BR = 400
NRB = N // BR
TW = 128
F32 = jnp.float32
BF16 = jnp.bfloat16
INT32 = jnp.int32

_INTERP = False


def _dot(a, w_bf16):
    return jnp.dot(a.astype(BF16), w_bf16, preferred_element_type=F32)


def _gelu(z):
    return jax.nn.gelu(z)


def _embed_body(x_ref, w_ref, b_ref, t_ref):
    xb = x_ref[0]
    h = _dot(xb[:, 3:7], w_ref[...]) + b_ref[...]
    zero1 = jnp.zeros((N, 1), F32)
    zpad = jnp.zeros((N, TW - 36), F32)
    t_ref[0] = jnp.concatenate([xb[:, 0:3], zero1, h, zpad], axis=1)


def _embed(x, w_bf16, b):
    return pl.pallas_call(
        _embed_body,
        grid=(2,),
        in_specs=[
            pl.BlockSpec((1, N, 7), lambda i: (i, 0, 0)),
            pl.BlockSpec((4, DH), lambda i: (0, 0)),
            pl.BlockSpec((DH,), lambda i: (0,)),
        ],
        out_specs=pl.BlockSpec((1, N, TW), lambda i: (i, 0, 0)),
        out_shape=jax.ShapeDtypeStruct((2, N, TW), F32),
        interpret=_INTERP,
    )(x, w_bf16, b)


def _knn_body(pos_ref, post_ref, idx_ref, d_ref):
    j = pl.program_id(1)
    b = pl.program_id(0)
    xi = pos_ref[0]
    pt = post_ref[0]
    dx = xi[:, 0:1] - pt[0:1, :]
    dy = xi[:, 1:2] - pt[1:2, :]
    dz = xi[:, 2:3] - pt[2:3, :]
    d2 = (dx * dx + dy * dy) + dz * dz
    col = lax.broadcasted_iota(INT32, (BR, N), 1)
    row = lax.broadcasted_iota(INT32, (BR, N), 0) + j * BR
    d_ref[...] = jnp.where(col == row, d2 + 1e9, d2)
    for t in range(K):
        d = d_ref[...]
        m = jnp.min(d, axis=1, keepdims=True)
        am = jnp.min(jnp.where(d == m, col, jnp.int32(2 ** 30)),
                     axis=1)
        idx_ref[0, :, t] = am + b * N
        d_ref[...] = jnp.where(col == am[:, None], jnp.float32(jnp.inf), d)


def _knn(pos4, post):
    return pl.pallas_call(
        _knn_body,
        grid=(2, NRB),
        in_specs=[
            pl.BlockSpec((1, BR, 4), lambda b, j: (b, j, 0)),
            pl.BlockSpec((1, 4, N), lambda b, j: (b, 0, 0)),
        ],
        out_specs=pl.BlockSpec((1, BR, K), lambda b, j: (b, j, 0)),
        out_shape=jax.ShapeDtypeStruct((2, N, K), INT32),
        scratch_shapes=[pltpu.VMEM((BR, N), F32)],
        interpret=_INTERP,
    )(pos4, post)


def _sc_gather(table, gidx):
    nrows = 2 * K * N
    per_w = nrows // 32
    ch = 1000
    nch = per_w // ch

    mesh = plsc.VectorSubcoreMesh(core_axis_name="c", subcore_axis_name="s")

    @functools.partial(
        pl.kernel,
        mesh=mesh,
        out_type=jax.ShapeDtypeStruct((nrows, TW), F32),
        scratch_types=[
            pltpu.VMEM((ch,), INT32),
            pltpu.VMEM((ch, TW), F32),
            pltpu.SemaphoreType.DMA,
        ],
    )
    def gather_k(table_hbm, gidx_hbm, out_hbm, idx_v, rows_v, sem):
        wid = lax.axis_index("s") * 2 + lax.axis_index("c")
        base = wid * per_w
        for c in range(nch):
            off = base + c * ch
            pltpu.sync_copy(gidx_hbm.at[pl.ds(off, ch)], idx_v)
            pltpu.async_copy(table_hbm.at[idx_v], rows_v, sem).wait()
            pltpu.sync_copy(rows_v, out_hbm.at[pl.ds(off, ch)])

    return gather_k(table, gidx)


def _dott(wt_bf16, x):
    return jnp.dot(wt_bf16, x.astype(BF16), preferred_element_type=F32)


def _step_body(t_ref, g_ref,
               we1, be1, we2, be2, we3, be3,
               wx1, bx1, wx2, bx2, wx3, bx3,
               wh1, bh1, wh2, bh2, wh3, bh3,
               tn_ref):
    tt = jnp.transpose(t_ref[0][:, 0:48])
    post = tt[0:4]
    ht = tt[4:36]
    gts = [jnp.transpose(g_ref[k][:, 0:48]) for k in range(K)]
    rs = []
    d2s = []
    for k in range(K):
        rt = post - gts[k][0:4]
        x2 = rt[0:1] * rt[0:1]
        y2 = rt[1:2] * rt[1:2]
        z2 = rt[2:3] * rt[2:3]
        rs.append(rt)
        d2s.append((x2 + y2) + z2)
    ang = jnp.concatenate(
        [d2s[k] * float(2.0 ** f) for k in range(K) for f in range(NF)],
        axis=0)
    sin_all = jnp.sin(ang)
    cos_all = jnp.cos(ang)
    agg = jnp.zeros((DH, BN), F32)
    acc_rw = jnp.zeros((4, BN), F32)
    for k in range(K):
        edge = jnp.concatenate(
            [ht, gts[k][4:36],
             sin_all[8 * k:8 * k + 8], cos_all[8 * k:8 * k + 8]],
            axis=0)
        z = _gelu(_dott(we1[...], edge) + be1[...])
        z = _gelu(_dott(we2[...], z) + be2[...])
        m = _gelu(_dott(we3[...], z) + be3[...])
        u = _gelu(_dott(wx1[...], m) + bx1[...])
        u = _gelu(_dott(wx2[...], u) + bx2[...])
        w = _dott(wx3[...], u) + bx3[...]
        agg = agg + m
        acc_rw = acc_rw + rs[k] * w
    posn = post + acc_rw * (1.0 / K)
    cat = jnp.concatenate([ht, agg], axis=0)
    z = _gelu(_dott(wh1[...], cat) + bh1[...])
    z = _gelu(_dott(wh2[...], z) + bh2[...])
    hn = _dott(wh3[...], z) + bh3[...]
    out = jnp.transpose(jnp.concatenate([posn, hn], axis=0))
    zpad = jnp.zeros((BN, TW - 36), F32)
    tn_ref[0] = jnp.concatenate([out, zpad], axis=1)


def _step(t, g, wflat):
    wspecs = [pl.BlockSpec(w.shape, functools.partial(
        lambda rank, b, j: (0,) * rank, w.ndim)) for w in wflat]
    return pl.pallas_call(
        _step_body,
        grid=(2, NB),
        in_specs=[
            pl.BlockSpec((1, BN, TW), lambda b, j: (b, j, 0)),
            pl.BlockSpec((K, BN, TW), lambda b, j: (b, j, 0)),
        ] + wspecs,
        out_specs=pl.BlockSpec((1, BN, TW), lambda b, j: (b, j, 0)),
        out_shape=jax.ShapeDtypeStruct((2, N, TW), F32),
        interpret=_INTERP,
    )(t, g, *wflat)


def _decode_body(t_ref, w_ref, b_ref, out_ref):
    tb = t_ref[0]
    feat = _dot(tb[:, 4:36], w_ref[...]) + b_ref[...]
    out_ref[0] = jnp.concatenate([tb[:, 0:3], feat], axis=1)


def _decode(t, w_bf16, b):
    return pl.pallas_call(
        _decode_body,
        grid=(2,),
        in_specs=[
            pl.BlockSpec((1, N, TW), lambda i: (i, 0, 0)),
            pl.BlockSpec((DH, 4), lambda i: (0, 0)),
            pl.BlockSpec((4,), lambda i: (0,)),
        ],
        out_specs=pl.BlockSpec((1, N, 7), lambda i: (i, 0, 0)),
        out_shape=jax.ShapeDtypeStruct((2, N, 7), F32),
        interpret=_INTERP,
    )(t, w_bf16, b)


def _step_weights(p):
    ws = []
    for name in ('phi_e', 'phi_x', 'phi_h'):
        for (w, b) in p[name]:
            ws.append(w.T.astype(BF16))
            ws.append(b.reshape(-1, 1))
    return ws


def kernel(x, params):
    pos = x[:, :, 0:3]
    pos4 = jnp.pad(pos, ((0, 0), (0, 0), (0, 1)))
    post = jnp.transpose(pos4, (0, 2, 1))

    t = _embed(x, params['W_embed'].astype(BF16), params['b_embed'])
    gidx = jnp.transpose(_knn(pos4, post), (0, 2, 1)).reshape(2 * K * N)

    for p in params['steps']:
        g = _sc_gather(t.reshape(2 * N, TW), gidx).reshape(2 * K, N, TW)
        t = _step(t, g, _step_weights(p))

    return _decode(t, params['W_dec'].astype(BF16), params['b_dec'])


# speedup vs baseline: 1.0994x; 1.0283x over previous
import functools

import jax
import jax.numpy as jnp
from jax import lax
from jax.experimental import pallas as pl
from jax.experimental.pallas import tpu as pltpu
from jax.experimental.pallas import tpu_sc as plsc

DH = 32
K = 16
NF = 8
N = 2000
BN = 1000
NB = N // BN
BR = 400
NRB = N // BR
TW = 128
F32 = jnp.float32
BF16 = jnp.bfloat16
INT32 = jnp.int32

_INTERP = False


def _dot(a, w_bf16):
    return jnp.dot(a.astype(BF16), w_bf16, preferred_element_type=F32)


def _gelu(z):
    return jax.nn.gelu(z)


def _embed_body(x_ref, w_ref, b_ref, t_ref):
    xb = x_ref[0]
    h = _dot(xb[:, 3:7], w_ref[...]) + b_ref[...]
    zero1 = jnp.zeros((N, 1), F32)
    zpad = jnp.zeros((N, TW - 36), F32)
    t_ref[0] = jnp.concatenate([xb[:, 0:3], zero1, h, zpad], axis=1)


def _embed(x, w_bf16, b):
    return pl.pallas_call(
        _embed_body,
        grid=(2,),
        in_specs=[
            pl.BlockSpec((1, N, 7), lambda i: (i, 0, 0)),
            pl.BlockSpec((4, DH), lambda i: (0, 0)),
            pl.BlockSpec((DH,), lambda i: (0,)),
        ],
        out_specs=pl.BlockSpec((1, N, TW), lambda i: (i, 0, 0)),
        out_shape=jax.ShapeDtypeStruct((2, N, TW), F32),
        interpret=_INTERP,
    )(x, w_bf16, b)


def _knn_body(pos_ref, post_ref, idx_ref, d_ref):
    j = pl.program_id(1)
    b = pl.program_id(0)
    xi = pos_ref[0]
    pt = post_ref[0]
    dx = xi[:, 0:1] - pt[0:1, :]
    dy = xi[:, 1:2] - pt[1:2, :]
    dz = xi[:, 2:3] - pt[2:3, :]
    d2 = (dx * dx + dy * dy) + dz * dz
    col = lax.broadcasted_iota(INT32, (BR, N), 1)
    row = lax.broadcasted_iota(INT32, (BR, N), 0) + j * BR
    d_ref[...] = jnp.where(col == row, d2 + 1e9, d2)
    for t in range(K):
        d = d_ref[...]
        m = jnp.min(d, axis=1, keepdims=True)
        am = jnp.min(jnp.where(d == m, col, jnp.int32(2 ** 30)),
                     axis=1)
        idx_ref[0, :, t] = am
        d_ref[...] = jnp.where(col == am[:, None], jnp.float32(jnp.inf), d)


def _knn(pos4, post):
    return pl.pallas_call(
        _knn_body,
        grid=(2, NRB),
        in_specs=[
            pl.BlockSpec((1, BR, 4), lambda b, j: (b, j, 0)),
            pl.BlockSpec((1, 4, N), lambda b, j: (b, 0, 0)),
        ],
        out_specs=pl.BlockSpec((1, BR, K), lambda b, j: (b, j, 0)),
        out_shape=jax.ShapeDtypeStruct((2, N, K), INT32),
        scratch_shapes=[pltpu.VMEM((BR, N), F32)],
        interpret=_INTERP,
    )(pos4, post)


def _sc_gather(table, gidx):
    nrows = K * N
    per_w = nrows // 32
    ch = 1000
    nch = per_w // ch

    mesh = plsc.VectorSubcoreMesh(core_axis_name="c", subcore_axis_name="s")

    @functools.partial(
        pl.kernel,
        mesh=mesh,
        out_type=jax.ShapeDtypeStruct((nrows, TW), F32),
        scratch_types=[
            pltpu.VMEM((ch,), INT32),
            pltpu.VMEM((ch, TW), F32),
            pltpu.SemaphoreType.DMA,
        ],
    )
    def gather_k(table_hbm, gidx_hbm, out_hbm, idx_v, rows_v, sem):
        wid = lax.axis_index("s") * 2 + lax.axis_index("c")
        base = wid * per_w
        for c in range(nch):
            off = base + c * ch
            pltpu.sync_copy(gidx_hbm.at[pl.ds(off, ch)], idx_v)
            pltpu.async_copy(table_hbm.at[idx_v], rows_v, sem).wait()
            pltpu.sync_copy(rows_v, out_hbm.at[pl.ds(off, ch)])

    return gather_k(table, gidx)


def _dott(wt_bf16, x):
    return jnp.dot(wt_bf16, x.astype(BF16), preferred_element_type=F32)


def _step_body(t_ref, g_ref,
               we1, be1, we2, be2, we3, be3,
               wx1, bx1, wx2, bx2, wx3, bx3,
               wh1, bh1, wh2, bh2, wh3, bh3,
               tn_ref):
    tt = jnp.transpose(t_ref[0][:, 0:48])
    post = tt[0:4]
    ht = tt[4:36]
    gts = [jnp.transpose(g_ref[k][:, 0:48]) for k in range(K)]
    rs = []
    d2s = []
    for k in range(K):
        rt = post - gts[k][0:4]
        x2 = rt[0:1] * rt[0:1]
        y2 = rt[1:2] * rt[1:2]
        z2 = rt[2:3] * rt[2:3]
        rs.append(rt)
        d2s.append((x2 + y2) + z2)
    ang = jnp.concatenate(
        [d2s[k] * float(2.0 ** f) for k in range(K) for f in range(NF)],
        axis=0)
    sin_all = jnp.sin(ang)
    cos_all = jnp.cos(ang)
    agg = jnp.zeros((DH, BN), F32)
    acc_rw = jnp.zeros((4, BN), F32)
    for k in range(K):
        edge = jnp.concatenate(
            [ht, gts[k][4:36],
             sin_all[8 * k:8 * k + 8], cos_all[8 * k:8 * k + 8]],
            axis=0)
        z = _gelu(_dott(we1[...], edge) + be1[...])
        z = _gelu(_dott(we2[...], z) + be2[...])
        m = _gelu(_dott(we3[...], z) + be3[...])
        u = _gelu(_dott(wx1[...], m) + bx1[...])
        u = _gelu(_dott(wx2[...], u) + bx2[...])
        w = _dott(wx3[...], u) + bx3[...]
        agg = agg + m
        acc_rw = acc_rw + rs[k] * w
    posn = post + acc_rw * (1.0 / K)
    cat = jnp.concatenate([ht, agg], axis=0)
    z = _gelu(_dott(wh1[...], cat) + bh1[...])
    z = _gelu(_dott(wh2[...], z) + bh2[...])
    hn = _dott(wh3[...], z) + bh3[...]
    out = jnp.transpose(jnp.concatenate([posn, hn], axis=0))
    zpad = jnp.zeros((BN, TW - 36), F32)
    tn_ref[0] = jnp.concatenate([out, zpad], axis=1)


def _step(t, g, wflat):
    wspecs = [pl.BlockSpec(w.shape, functools.partial(
        lambda rank, b, j: (0,) * rank, w.ndim)) for w in wflat]
    return pl.pallas_call(
        _step_body,
        grid=(1, NB),
        in_specs=[
            pl.BlockSpec((1, BN, TW), lambda b, j: (b, j, 0)),
            pl.BlockSpec((K, BN, TW), lambda b, j: (b, j, 0)),
        ] + wspecs,
        out_specs=pl.BlockSpec((1, BN, TW), lambda b, j: (b, j, 0)),
        out_shape=jax.ShapeDtypeStruct((1, N, TW), F32),
        interpret=_INTERP,
    )(t, g, *wflat)


def _decode_body(t_ref, w_ref, b_ref, out_ref):
    tb = t_ref[0]
    feat = _dot(tb[:, 4:36], w_ref[...]) + b_ref[...]
    out_ref[0] = jnp.concatenate([tb[:, 0:3], feat], axis=1)


def _decode(t, w_bf16, b):
    return pl.pallas_call(
        _decode_body,
        grid=(2,),
        in_specs=[
            pl.BlockSpec((1, N, TW), lambda i: (i, 0, 0)),
            pl.BlockSpec((DH, 4), lambda i: (0, 0)),
            pl.BlockSpec((4,), lambda i: (0,)),
        ],
        out_specs=pl.BlockSpec((1, N, 7), lambda i: (i, 0, 0)),
        out_shape=jax.ShapeDtypeStruct((2, N, 7), F32),
        interpret=_INTERP,
    )(t, w_bf16, b)


def _step_weights(p):
    ws = []
    for name in ('phi_e', 'phi_x', 'phi_h'):
        for (w, b) in p[name]:
            ws.append(w.T.astype(BF16))
            ws.append(b.reshape(-1, 1))
    return ws


def kernel(x, params):
    pos = x[:, :, 0:3]
    pos4 = jnp.pad(pos, ((0, 0), (0, 0), (0, 1)))
    post = jnp.transpose(pos4, (0, 2, 1))

    t = _embed(x, params['W_embed'].astype(BF16), params['b_embed'])
    idx = _knn(pos4, post)
    gidxs = [jnp.transpose(idx[b], (1, 0)).reshape(K * N) for b in range(2)]
    ts = [t[b:b + 1] for b in range(2)]

    for p in params['steps']:
        wflat = _step_weights(p)
        gs = [_sc_gather(ts[b].reshape(N, TW), gidxs[b]).reshape(K, N, TW)
              for b in range(2)]
        ts = [_step(ts[b], gs[b], wflat) for b in range(2)]

    t = jnp.concatenate(ts, axis=0)
    return _decode(t, params['W_dec'].astype(BF16), params['b_dec'])
